# Initial kernel scaffold; baseline (speedup 1.0000x reference)
import jax
import jax.numpy as jnp
from jax.experimental import pallas as pl


def kernel(h, r, pos_t, neg_t, entity_user_embed, relation_embed, trans_M):
    raise NotImplementedError("write your pallas kernel here")


# speedup vs baseline: 1.1996x; 1.1996x over previous
import functools

import jax
import jax.numpy as jnp
from jax import lax
from jax.experimental import pallas as pl
from jax.experimental.pallas import tpu as pltpu
from jax.experimental.pallas import tpu_sc as plsc

_B = 16384
_D = 64
_R = 64
_NREL = 32
_L2_LAMBDA = 1e-05

_NC = 2
_NS = 16
_NW = _NC * _NS
_TOT = 3 * _B
_BPW = _TOT // _NW
_CH = 128
_NCH = _BPW // _CH

_BLK = 512
_NBLK = _B // _BLK


def _sc_gather_body(table_hbm, idx_hbm, out_hbm, idx_v, rows_v, sem):
    wid = lax.axis_index("s") * _NC + lax.axis_index("c")
    pltpu.sync_copy(idx_hbm.at[wid], idx_v)
    copies = [
        pltpu.async_copy(table_hbm.at[idx_v.at[j]], rows_v.at[j], sem)
        for j in range(_NCH)
    ]
    for c in copies:
        c.wait()
    pltpu.sync_copy(rows_v, out_hbm.at[wid])


@functools.cache
def _sc_gather():
    return pl.kernel(
        _sc_gather_body,
        out_type=jax.ShapeDtypeStruct((_NW, _NCH, _CH, _D), jnp.float32),
        mesh=plsc.VectorSubcoreMesh(core_axis_name="c", subcore_axis_name="s",
                                    num_cores=_NC, num_subcores=_NS),
        scratch_types=[
            pltpu.VMEM((_NCH, _CH), jnp.int32),
            pltpu.VMEM((_NCH, _CH, _D), jnp.float32),
            pltpu.SemaphoreType.DMA,
        ],
        compiler_params=pltpu.CompilerParams(use_tc_tiling_on_sc=False),
    )


def _tc_loss_body(r_ref, xh_ref, xp_ref, xn_ref, mflat_ref, rel_ref, out_ref):
    i = pl.program_id(0)
    r = r_ref[0, 0, :]

    col32 = lax.broadcasted_iota(jnp.int32, (_BLK, _NREL), 1)
    onehot = (r[:, None] == col32).astype(jnp.float32)
    r_emb = jnp.dot(onehot, rel_ref[...],
                    preferred_element_type=jnp.float32)

    colk = lax.broadcasted_iota(jnp.int32, (_BLK, _NREL * _D), 1) // _D
    m = r[:, None] == colk
    mflat = mflat_ref[...]

    def proj(x_ref):
        x = x_ref[...]
        tiled = jnp.concatenate([x] * _NREL, axis=1)
        u = jnp.where(m, tiled, 0.0)
        return jnp.dot(u, mflat, preferred_element_type=jnp.float32)

    mh = proj(xh_ref)
    mp = proj(xp_ref)
    mn = proj(xn_ref)

    pos = jnp.sum((mh + r_emb - mp) ** 2, axis=1, keepdims=True)
    neg = jnp.sum((mh + r_emb - mn) ** 2, axis=1, keepdims=True)
    d = neg - pos
    kg = jnp.sum(jnp.maximum(-d, 0.0) + jnp.log(1.0 + jnp.exp(-jnp.abs(d))))
    l2 = 0.5 * (jnp.sum(mh * mh) + jnp.sum(r_emb * r_emb)
                + jnp.sum(mp * mp) + jnp.sum(mn * mn))
    part = jnp.reshape((kg + _L2_LAMBDA * l2) * (1.0 / _B), (1, 1))

    @pl.when(i == 0)
    def _():
        out_ref[...] = jnp.zeros((1, 1), jnp.float32)

    out_ref[...] += part


def kernel(h, r, pos_t, neg_t, entity_user_embed, relation_embed, trans_M):
    idx = jnp.concatenate([h, pos_t, neg_t]).astype(jnp.int32)
    idx = idx.reshape(_NW, _NCH, _CH)
    gathered = _sc_gather()(entity_user_embed, idx)
    gathered = gathered.reshape(_TOT, _D)

    r3 = r.astype(jnp.int32).reshape(_NBLK, 1, _BLK)
    mflat = trans_M.reshape(_NREL * _D, _R)

    out = pl.pallas_call(
        _tc_loss_body,
        grid=(_NBLK,),
        in_specs=[
            pl.BlockSpec((1, 1, _BLK), lambda i: (i, 0, 0)),
            pl.BlockSpec((_BLK, _D), lambda i: (i, 0)),
            pl.BlockSpec((_BLK, _D), lambda i: (i + _NBLK, 0)),
            pl.BlockSpec((_BLK, _D), lambda i: (i + 2 * _NBLK, 0)),
            pl.BlockSpec((_NREL * _D, _R), lambda i: (0, 0)),
            pl.BlockSpec((_NREL, _R), lambda i: (0, 0)),
        ],
        out_specs=pl.BlockSpec((1, 1), lambda i: (0, 0)),
        out_shape=jax.ShapeDtypeStruct((1, 1), jnp.float32),
    )(r3, gathered, gathered, gathered, mflat, relation_embed)
    return out[0, 0]
